# Initial kernel scaffold; baseline (speedup 1.0000x reference)
import jax
import jax.numpy as jnp
from jax.experimental import pallas as pl


def kernel(x, pos_embedding_weight):
    raise NotImplementedError("write your pallas kernel here")


# speedup vs baseline: 3.4220x; 3.4220x over previous
import functools

import jax
import jax.numpy as jnp
from jax import lax
from jax.experimental import pallas as pl
from jax.experimental.pallas import tpu as pltpu
from jax.experimental.pallas import tpu_sc as plsc

_NC = 2
_NS = 16
_NW = _NC * _NS


def _make_sc_broadcast(batch: int, rows: int, dim: int):
    rows_per_w = rows // _NW
    chunk = 32
    nchunk = rows_per_w // chunk
    mesh = plsc.VectorSubcoreMesh(core_axis_name="c", subcore_axis_name="s")

    @functools.partial(
        pl.kernel,
        out_type=jax.ShapeDtypeStruct((batch, rows, dim), jnp.float32),
        mesh=mesh,
        scratch_types=[
            pltpu.VMEM((2, chunk, dim), jnp.float32),
            pltpu.SemaphoreType.DMA,
        ],
    )
    def sc_broadcast(table_hbm, out_hbm, buf, sem):
        wid = lax.axis_index("s") * _NC + lax.axis_index("c")
        base = wid * rows_per_w
        for c in range(nchunk):
            r0 = base + c * chunk
            slot = c % 2
            pltpu.sync_copy(table_hbm.at[pl.ds(r0, chunk)], buf.at[slot])
            for b in range(batch):
                pltpu.sync_copy(buf.at[slot], out_hbm.at[b, pl.ds(r0, chunk)])

    return sc_broadcast


def kernel(x, pos_embedding_weight):
    batch, seq_len = x.shape
    rows, dim = pos_embedding_weight.shape
    fn = _make_sc_broadcast(batch, rows, dim)
    return fn(pos_embedding_weight)


# speedup vs baseline: 3.5221x; 1.0293x over previous
import functools

import jax
import jax.numpy as jnp
from jax import lax
from jax.experimental import pallas as pl
from jax.experimental.pallas import tpu as pltpu
from jax.experimental.pallas import tpu_sc as plsc

_NC = 2
_NS = 16
_NW = _NC * _NS


def _make_sc_broadcast(batch: int, rows: int, dim: int):
    rows_per_w = rows // _NW
    chunk = 32
    nchunk = rows_per_w // chunk
    mesh = plsc.VectorSubcoreMesh(core_axis_name="c", subcore_axis_name="s")

    @functools.partial(
        pl.kernel,
        out_type=jax.ShapeDtypeStruct((batch, rows, dim), jnp.float32),
        mesh=mesh,
        scratch_types=[
            pltpu.VMEM((2, chunk, dim), jnp.float32),
            pltpu.SemaphoreType.DMA,
            pltpu.SemaphoreType.DMA,
            pltpu.SemaphoreType.DMA,
            pltpu.SemaphoreType.DMA,
        ],
    )
    def sc_broadcast(table_hbm, out_hbm, buf, rsem0, rsem1, wsem0, wsem1):
        wid = lax.axis_index("s") * _NC + lax.axis_index("c")
        base = wid * rows_per_w
        rsems = (rsem0, rsem1)
        wsems = (wsem0, wsem1)

        def start_read(c):
            r0 = base + c * chunk
            s = c % 2
            return pltpu.async_copy(
                table_hbm.at[pl.ds(r0, chunk)], buf.at[s], rsems[s]
            )

        reads = [None] * nchunk
        writes = [None] * nchunk
        reads[0] = start_read(0)
        for c in range(nchunk):
            s = c % 2
            if c + 1 < nchunk:
                if c >= 1:
                    for h in writes[c - 1]:
                        h.wait()
                reads[c + 1] = start_read(c + 1)
            reads[c].wait()
            r0 = base + c * chunk
            writes[c] = [
                pltpu.async_copy(buf.at[s], out_hbm.at[b, pl.ds(r0, chunk)], wsems[s])
                for b in range(batch)
            ]
        for c in (nchunk - 2, nchunk - 1):
            for h in writes[c]:
                h.wait()

    return sc_broadcast


def kernel(x, pos_embedding_weight):
    batch, seq_len = x.shape
    rows, dim = pos_embedding_weight.shape
    fn = _make_sc_broadcast(batch, rows, dim)
    return fn(pos_embedding_weight)
